# baseline (speedup 1.0000x reference)
import dataclasses
import functools

import jax
import jax.numpy as jnp
from jax import lax
from jax.experimental import pallas as pl
from jax.experimental.pallas import tpu as pltpu
from jax.experimental.pallas import tpu_sc as plsc

N = 50000
E = 800000
H = 64
G = 512

NW = 32
NPAD = 50176
NR = 392
SL = NPAD // NW
E_PAD = 802816
CH = 6272
W_AC = E_PAD // NW
NCH_AC = W_AC // CH
W_E = E_PAD // 16
NCH_E = W_E // CH
GP = 640

_mesh = plsc.VectorSubcoreMesh(core_axis_name="c", subcore_axis_name="s")
_f32 = jnp.float32

_sc_params = pltpu.CompilerParams()
if "needs_layout_passes" in pltpu.CompilerParams.__dataclass_fields__:
    _sc_params = dataclasses.replace(_sc_params, needs_layout_passes=False)


def _wid():
    return lax.axis_index("c") * 16 + lax.axis_index("s")


def _zero(buf, n):
    zs = jnp.zeros((16,), dtype=_f32)

    @pl.loop(0, n, step=16)
    def _z(i):
        buf[pl.ds(i, 16)] = zs


def _deg_body(dst_hbm, out_hbm, acc_v, dst_v0, dst_v1,
              sem0, sem1):
    wid = _wid()
    base = wid * W_AC
    ones = jnp.full((16,), 1.0, dtype=_f32)
    bufs = (dst_v0, dst_v1)
    sems = (sem0, sem1)

    def start(b, off):
        pltpu.async_copy(dst_hbm.at[pl.ds(off, CH)], bufs[b], sems[b])

    def wait(b):
        pltpu.make_async_copy(dst_hbm.at[pl.ds(0, CH)], bufs[b],
                              sems[b]).wait()

    def scat(b):
        @pl.loop(0, CH, step=128)
        def _inner(i):
            for u in range(8):
                di = bufs[b][pl.ds(i + u * 16, 16)]
                plsc.addupdate_scatter(acc_v, [di], ones)

    start(0, base)
    _zero(acc_v, NPAD)

    @pl.loop(0, NCH_AC, step=2)
    def _chunks(j):
        start(1, base + (j + 1) * CH)
        wait(0)
        scat(0)

        @pl.when(j + 2 < NCH_AC)
        def _():
            start(0, base + (j + 2) * CH)

        wait(1)
        scat(1)

    pltpu.sync_copy(acc_v, out_hbm.at[pl.ds(wid * NPAD, NPAD)])


def _sc_deg(dstp):
    fn = pl.kernel(
        _deg_body,
        out_type=jax.ShapeDtypeStruct((NW * NPAD,), _f32),
        mesh=_mesh,
        compiler_params=_sc_params,
        scratch_types=[
            pltpu.VMEM((NPAD,), _f32),
            pltpu.VMEM((CH,), jnp.int32),
            pltpu.VMEM((CH,), jnp.int32),
            pltpu.SemaphoreType.DMA,
            pltpu.SemaphoreType.DMA,
        ],
    )
    return fn(dstp)


def _gs_loop(src_hbm, dst_hbm, out_hbm, t_v, acc_v, sbufs, dbufs, sems,
             wid, base, nch):
    def start(b, off):
        pltpu.async_copy(src_hbm.at[pl.ds(off, CH)], sbufs[b], sems[b])
        pltpu.async_copy(dst_hbm.at[pl.ds(off, CH)], dbufs[b], sems[b])

    def wait(b):
        pltpu.make_async_copy(src_hbm.at[pl.ds(0, CH)], sbufs[b],
                              sems[b]).wait()
        pltpu.make_async_copy(dst_hbm.at[pl.ds(0, CH)], dbufs[b],
                              sems[b]).wait()

    def scat(b):
        @pl.loop(0, CH, step=128)
        def _inner(i):
            for u in range(8):
                si = sbufs[b][pl.ds(i + u * 16, 16)]
                di = dbufs[b][pl.ds(i + u * 16, 16)]
                vals = plsc.load_gather(t_v, [si])
                plsc.addupdate_scatter(acc_v, [di], vals)

    @pl.loop(0, nch, step=2)
    def _chunks(j):
        start(1, base + (j + 1) * CH)
        wait(0)
        scat(0)

        @pl.when(j + 2 < nch)
        def _():
            start(0, base + (j + 2) * CH)

        wait(1)
        scat(1)

    pltpu.sync_copy(acc_v, out_hbm.at[pl.ds(wid * NPAD, NPAD)])


def _gs_body(src_hbm, dst_hbm, t_hbm, out_hbm,
             t_v, acc_v, src_v0, dst_v0, src_v1, dst_v1, t_sh, semt,
             sem0, sem1):
    c = lax.axis_index("c")
    s = lax.axis_index("s")
    wid = c * 16 + s
    base = wid * W_AC

    @pl.when(s == 0)
    def _():
        pltpu.async_copy(t_hbm, t_sh, semt)

    pltpu.async_copy(src_hbm.at[pl.ds(base, CH)], src_v0, sem0)
    pltpu.async_copy(dst_hbm.at[pl.ds(base, CH)], dst_v0, sem0)
    _zero(acc_v, NPAD)

    @pl.when(s == 0)
    def _():
        pltpu.make_async_copy(t_hbm, t_sh, semt).wait()

    plsc.subcore_barrier()
    pltpu.sync_copy(t_sh, t_v)
    _gs_loop(src_hbm, dst_hbm, out_hbm, t_v, acc_v,
             (src_v0, src_v1), (dst_v0, dst_v1), (sem0, sem1),
             wid, base, NCH_AC)


def _sc_gs(srcp, dstp, t1d):
    fn = pl.kernel(
        _gs_body,
        out_type=jax.ShapeDtypeStruct((NW * NPAD,), _f32),
        mesh=_mesh,
        compiler_params=_sc_params,
        scratch_types=[
            pltpu.VMEM((NPAD,), _f32),
            pltpu.VMEM((NPAD,), _f32),
            pltpu.VMEM((CH,), jnp.int32),
            pltpu.VMEM((CH,), jnp.int32),
            pltpu.VMEM((CH,), jnp.int32),
            pltpu.VMEM((CH,), jnp.int32),
            pltpu.VMEM_SHARED((NPAD,), _f32),
            pltpu.SemaphoreType.DMA,
            pltpu.SemaphoreType.DMA,
            pltpu.SemaphoreType.DMA,
        ],
    )
    return fn(srcp, dstp, t1d)


def _ac_body(src_hbm, dst_hbm, tatc_hbm, out_hbm,
             t_v, acc_v, src_v0, dst_v0, src_v1, dst_v1, t_sh, semt,
             sem0, sem1):
    c = lax.axis_index("c")
    s = lax.axis_index("s")
    wid = c * 16 + s
    base = s * W_E

    @pl.when(s == 0)
    def _():
        pltpu.async_copy(tatc_hbm.at[pl.ds(c * NPAD, NPAD)], t_sh, semt)

    pltpu.async_copy(src_hbm.at[pl.ds(base, CH)], src_v0, sem0)
    pltpu.async_copy(dst_hbm.at[pl.ds(base, CH)], dst_v0, sem0)
    _zero(acc_v, NPAD)

    @pl.when(s == 0)
    def _():
        pltpu.make_async_copy(tatc_hbm.at[pl.ds(0, NPAD)], t_sh, semt).wait()

    plsc.subcore_barrier()
    pltpu.sync_copy(t_sh, t_v)
    _gs_loop(src_hbm, dst_hbm, out_hbm, t_v, acc_v,
             (src_v0, src_v1), (dst_v0, dst_v1), (sem0, sem1),
             wid, base, NCH_E)


def _sc_ac(srcp, dstp, tatc):
    fn = pl.kernel(
        _ac_body,
        out_type=jax.ShapeDtypeStruct((NW * NPAD,), _f32),
        mesh=_mesh,
        compiler_params=_sc_params,
        scratch_types=[
            pltpu.VMEM((NPAD,), _f32),
            pltpu.VMEM((NPAD,), _f32),
            pltpu.VMEM((CH,), jnp.int32),
            pltpu.VMEM((CH,), jnp.int32),
            pltpu.VMEM((CH,), jnp.int32),
            pltpu.VMEM((CH,), jnp.int32),
            pltpu.VMEM_SHARED((NPAD,), _f32),
            pltpu.SemaphoreType.DMA,
            pltpu.SemaphoreType.DMA,
            pltpu.SemaphoreType.DMA,
        ],
    )
    return fn(srcp, dstp, tatc)


def _pool_body(part_hbm, tatc_hbm, dinv_hbm, batch_hbm, out_hbm,
               p_v, ta_v, tc_v, dinv_v, b_v, accA, accC, accN,
               semp, semm):
    wid = _wid()
    lo = wid * SL
    zs = jnp.zeros((16,), dtype=_f32)

    @pl.loop(0, NW)
    def _ld(r):
        pltpu.async_copy(part_hbm.at[pl.ds(r * NPAD + lo, SL)],
                         p_v.at[pl.ds(r * SL, SL)], semp)

    pltpu.async_copy(tatc_hbm.at[pl.ds(lo, SL)], ta_v, semm)
    pltpu.async_copy(tatc_hbm.at[pl.ds(NPAD + lo, SL)], tc_v, semm)
    pltpu.async_copy(dinv_hbm.at[pl.ds(lo, SL)], dinv_v, semm)
    pltpu.async_copy(batch_hbm.at[pl.ds(lo, SL)], b_v, semm)

    @pl.loop(0, GP, step=16)
    def _z(i):
        accA[pl.ds(i, 16)] = zs
        accC[pl.ds(i, 16)] = zs
        accN[pl.ds(i, 16)] = zs

    @pl.loop(0, NW)
    def _wt(r):
        pltpu.make_async_copy(part_hbm.at[pl.ds(lo, SL)],
                              p_v.at[pl.ds(0, SL)], semp).wait()

    pltpu.make_async_copy(dinv_hbm.at[pl.ds(lo, SL)], ta_v, semm).wait()
    pltpu.make_async_copy(dinv_hbm.at[pl.ds(lo, SL)], tc_v, semm).wait()
    pltpu.make_async_copy(dinv_hbm.at[pl.ds(lo, SL)], dinv_v, semm).wait()
    pltpu.make_async_copy(batch_hbm.at[pl.ds(lo, SL)], b_v, semm).wait()
    ones = jnp.full((16,), 1.0, dtype=_f32)

    @pl.loop(0, SL, step=16)
    def _inner(i):
        sl = pl.ds(i, 16)
        Ap = p_v[pl.ds(i, 16)]
        Cp = p_v[pl.ds(16 * SL + i, 16)]
        for r in range(1, 16):
            Ap = Ap + p_v[pl.ds(r * SL + i, 16)]
            Cp = Cp + p_v[pl.ds((16 + r) * SL + i, 16)]
        dv = dinv_v[sl]
        Av = dv * (Ap + ta_v[sl])
        Cv = dv * (Cp + tc_v[sl])
        bi = b_v[sl]
        plsc.addupdate_scatter(accA, [bi], Av)
        plsc.addupdate_scatter(accC, [bi], Cv)
        plsc.addupdate_scatter(accN, [bi], ones)

    base3 = wid * 3 * GP
    pltpu.sync_copy(accA, out_hbm.at[pl.ds(base3, GP)])
    pltpu.sync_copy(accC, out_hbm.at[pl.ds(base3 + GP, GP)])
    pltpu.sync_copy(accN, out_hbm.at[pl.ds(base3 + 2 * GP, GP)])


def _sc_pool(part, tatc, dinv1d, batchp):
    fn = pl.kernel(
        _pool_body,
        out_type=jax.ShapeDtypeStruct((NW * 3 * GP,), _f32),
        mesh=_mesh,
        compiler_params=_sc_params,
        scratch_types=[
            pltpu.VMEM((NW * SL,), _f32),
            pltpu.VMEM((SL,), _f32),
            pltpu.VMEM((SL,), _f32),
            pltpu.VMEM((SL,), _f32),
            pltpu.VMEM((SL,), jnp.int32),
            pltpu.VMEM((GP,), _f32),
            pltpu.VMEM((GP,), _f32),
            pltpu.VMEM((GP,), _f32),
            pltpu.SemaphoreType.DMA,
            pltpu.SemaphoreType.DMA,
        ],
    )
    return fn(part, tatc, dinv1d, batchp)


def _tcB_body(part_ref, x_ref, dinv_ref, t_ref):
    deg = jnp.sum(part_ref[...], axis=0) + 1.0
    dinv = lax.rsqrt(deg)
    dinv_ref[...] = dinv
    t_ref[...] = x_ref[...] * dinv


def _tc_b(part3, x2):
    return pl.pallas_call(
        _tcB_body,
        out_shape=[
            jax.ShapeDtypeStruct((NR, 128), _f32),
            jax.ShapeDtypeStruct((NR, 128), _f32),
        ],
    )(part3, x2)


def _tcD_body(part_ref, t_ref, dinv_ref, tatc_ref):
    sp = jnp.sum(part_ref[...], axis=0)
    dinv = dinv_ref[...]
    s = dinv * (sp + t_ref[...])
    tatc_ref[0] = dinv * jnp.maximum(s, 0.0)
    tatc_ref[1] = dinv * jnp.maximum(-s, 0.0)


def _tc_d(part3, t2, dinv2):
    return pl.pallas_call(
        _tcD_body,
        out_shape=jax.ShapeDtypeStruct((2, NR, 128), _f32),
    )(part3, t2, dinv2)


def _dot(a, b, dims):
    return lax.dot_general(a, b, (dims, ((), ())),
                           precision=lax.Precision.HIGHEST,
                           preferred_element_type=_f32)


def _tcF_body(pp_ref, W1_ref, W2_ref, Wc1_ref, Wc2_ref, b2_ref, bc1_ref,
              bc2_ref, out_ref):
    p3 = jnp.sum(pp_ref[...], axis=0)
    sumA = p3[0:1, 0:G]
    sumC = p3[1:2, 0:G]
    cnt = p3[2:3, 0:G]
    invc = 1.0 / jnp.maximum(cnt, 1.0)
    Abar = sumA * invc
    Cbar = sumC * invc
    m = jnp.minimum(cnt, 1.0)

    u = jnp.maximum(W1_ref[...], 0.0)
    v = jnp.maximum(-W1_ref[...], 0.0)
    p = _dot(u, W2_ref[...], ((1,), (0,)))
    q = _dot(v, W2_ref[...], ((1,), (0,)))
    Pc = _dot(Wc1_ref[...], p, ((0,), (1,)))
    Qc = _dot(Wc1_ref[...], q, ((0,), (1,)))
    Rc = _dot(Wc1_ref[...], b2_ref[...], ((0,), (1,)))
    zT = Pc * Abar + Qc * Cbar + Rc * m + bc1_ref[...]
    zT = jnp.maximum(zT, 0.0)
    o = _dot(Wc2_ref[...], zT, ((0,), (0,)))
    o = o + bc2_ref[...]
    out_ref[...] = 1.0 / (1.0 + jnp.exp(-o))


def _tc_f(pp, W1, W2, Wc1, Wc2, b2, bc1, bc2):
    return pl.pallas_call(
        _tcF_body,
        out_shape=jax.ShapeDtypeStruct((1, G), _f32),
    )(pp, W1, W2, Wc1, Wc2, b2.reshape(1, H), bc1.reshape(32, 1),
      bc2.reshape(1, 1))


def kernel(x, edge_index, batch, W1, b1, W2, b2, Wc1, bc1, Wc2, bc2):
    i32 = jnp.int32
    pad = jnp.full((E_PAD - E,), N, i32)
    srcp = jnp.concatenate([edge_index[0].astype(i32), pad])
    dstp = jnp.concatenate([edge_index[1].astype(i32), pad])
    xpad = jnp.pad(x[:, 0], (0, NPAD - N))
    batchp = jnp.pad(batch.astype(i32), (0, NPAD - N), constant_values=G)
    degp = _sc_deg(dstp)
    dinv2, t2 = _tc_b(degp.reshape(NW, NR, 128), xpad.reshape(NR, 128))
    t1d = t2.reshape(NPAD)
    dinv1d = dinv2.reshape(NPAD)

    spp = _sc_gs(srcp, dstp, t1d)
    tatc3 = _tc_d(spp.reshape(NW, NR, 128), t2, dinv2)
    tatc = tatc3.reshape(2 * NPAD)

    acp = _sc_ac(srcp, dstp, tatc)
    pooled = _sc_pool(acp, tatc, dinv1d, batchp)

    out = _tc_f(pooled.reshape(NW, 3, GP), W1, W2, Wc1, Wc2, b2, bc1, bc2)
    return out.reshape(G, 1)


# speedup vs baseline: 1.0189x; 1.0189x over previous
import dataclasses
import functools

import jax
import jax.numpy as jnp
from jax import lax
from jax.experimental import pallas as pl
from jax.experimental.pallas import tpu as pltpu
from jax.experimental.pallas import tpu_sc as plsc

N = 50000
E = 800000
H = 64
G = 512

NW = 32
NPAD = 50176
NR = 392
SL = NPAD // NW
E_PAD = 802816
CH = 6272
W_AC = E_PAD // NW
NCH_AC = W_AC // CH
W_E = E_PAD // 16
NCH_E = W_E // CH
GP = 640

_mesh = plsc.VectorSubcoreMesh(core_axis_name="c", subcore_axis_name="s")
_f32 = jnp.float32

_sc_params = pltpu.CompilerParams()
if "needs_layout_passes" in pltpu.CompilerParams.__dataclass_fields__:
    _sc_params = dataclasses.replace(_sc_params, needs_layout_passes=False)


def _wid():
    return lax.axis_index("c") * 16 + lax.axis_index("s")


def _zero(buf, n):
    zs = jnp.zeros((16,), dtype=_f32)

    @pl.loop(0, n, step=16)
    def _z(i):
        buf[pl.ds(i, 16)] = zs


def _deg_body(dst_hbm, out_hbm, acc_v, dst_v0, dst_v1,
              sem0, sem1):
    wid = _wid()
    base = wid * W_AC
    ones = jnp.full((16,), 1.0, dtype=_f32)
    bufs = (dst_v0, dst_v1)
    sems = (sem0, sem1)

    def start(b, off):
        pltpu.async_copy(dst_hbm.at[pl.ds(off, CH)], bufs[b], sems[b])

    def wait(b):
        pltpu.make_async_copy(dst_hbm.at[pl.ds(0, CH)], bufs[b],
                              sems[b]).wait()

    def scat(b):
        @pl.loop(0, CH, step=128)
        def _inner(i):
            for u in range(8):
                di = bufs[b][pl.ds(i + u * 16, 16)]
                plsc.addupdate_scatter(acc_v, [di], ones)

    start(0, base)
    _zero(acc_v, NPAD)

    @pl.loop(0, NCH_AC, step=2)
    def _chunks(j):
        start(1, base + (j + 1) * CH)
        wait(0)
        scat(0)

        @pl.when(j + 2 < NCH_AC)
        def _():
            start(0, base + (j + 2) * CH)

        wait(1)
        scat(1)

    pltpu.sync_copy(acc_v, out_hbm.at[pl.ds(wid * NPAD, NPAD)])


def _sc_deg(dstp):
    fn = pl.kernel(
        _deg_body,
        out_type=jax.ShapeDtypeStruct((NW * NPAD,), _f32),
        mesh=_mesh,
        compiler_params=_sc_params,
        scratch_types=[
            pltpu.VMEM((NPAD,), _f32),
            pltpu.VMEM((CH,), jnp.int32),
            pltpu.VMEM((CH,), jnp.int32),
            pltpu.SemaphoreType.DMA,
            pltpu.SemaphoreType.DMA,
        ],
    )
    return fn(dstp)


def _gs_loop(src_hbm, dst_hbm, out_hbm, t_v, acc_v, sbufs, dbufs, sems,
             wid, base, nch):
    def start(b, off):
        pltpu.async_copy(src_hbm.at[pl.ds(off, CH)], sbufs[b], sems[b])
        pltpu.async_copy(dst_hbm.at[pl.ds(off, CH)], dbufs[b], sems[b])

    def wait(b):
        pltpu.make_async_copy(src_hbm.at[pl.ds(0, CH)], sbufs[b],
                              sems[b]).wait()
        pltpu.make_async_copy(dst_hbm.at[pl.ds(0, CH)], dbufs[b],
                              sems[b]).wait()

    def scat(b):
        @pl.loop(0, CH, step=128)
        def _inner(i):
            for u in range(8):
                si = sbufs[b][pl.ds(i + u * 16, 16)]
                di = dbufs[b][pl.ds(i + u * 16, 16)]
                vals = plsc.load_gather(t_v, [si])
                plsc.addupdate_scatter(acc_v, [di], vals)

    @pl.loop(0, nch, step=2)
    def _chunks(j):
        start(1, base + (j + 1) * CH)
        wait(0)
        scat(0)

        @pl.when(j + 2 < nch)
        def _():
            start(0, base + (j + 2) * CH)

        wait(1)
        scat(1)

    pltpu.sync_copy(acc_v, out_hbm.at[pl.ds(wid * NPAD, NPAD)])


def _gs_body(src_hbm, dst_hbm, t_hbm, out_hbm,
             t_v, acc_v, src_v0, dst_v0, src_v1, dst_v1, semt, sem0, sem1):
    wid = _wid()
    base = wid * W_AC
    pltpu.async_copy(t_hbm, t_v, semt)
    pltpu.async_copy(src_hbm.at[pl.ds(base, CH)], src_v0, sem0)
    pltpu.async_copy(dst_hbm.at[pl.ds(base, CH)], dst_v0, sem0)
    _zero(acc_v, NPAD)
    pltpu.make_async_copy(t_hbm, t_v, semt).wait()
    _gs_loop(src_hbm, dst_hbm, out_hbm, t_v, acc_v,
             (src_v0, src_v1), (dst_v0, dst_v1), (sem0, sem1),
             wid, base, NCH_AC)


def _sc_gs(srcp, dstp, t1d):
    fn = pl.kernel(
        _gs_body,
        out_type=jax.ShapeDtypeStruct((NW * NPAD,), _f32),
        mesh=_mesh,
        compiler_params=_sc_params,
        scratch_types=[
            pltpu.VMEM((NPAD,), _f32),
            pltpu.VMEM((NPAD,), _f32),
            pltpu.VMEM((CH,), jnp.int32),
            pltpu.VMEM((CH,), jnp.int32),
            pltpu.VMEM((CH,), jnp.int32),
            pltpu.VMEM((CH,), jnp.int32),
            pltpu.SemaphoreType.DMA,
            pltpu.SemaphoreType.DMA,
            pltpu.SemaphoreType.DMA,
        ],
    )
    return fn(srcp, dstp, t1d)


def _ac_body(src_hbm, dst_hbm, tatc_hbm, out_hbm,
             t_v, acc_v, src_v0, dst_v0, src_v1, dst_v1, semt, sem0, sem1):
    c = lax.axis_index("c")
    s = lax.axis_index("s")
    wid = c * 16 + s
    base = s * W_E
    pltpu.async_copy(tatc_hbm.at[pl.ds(c * NPAD, NPAD)], t_v, semt)
    pltpu.async_copy(src_hbm.at[pl.ds(base, CH)], src_v0, sem0)
    pltpu.async_copy(dst_hbm.at[pl.ds(base, CH)], dst_v0, sem0)
    _zero(acc_v, NPAD)
    pltpu.make_async_copy(tatc_hbm.at[pl.ds(0, NPAD)], t_v, semt).wait()
    _gs_loop(src_hbm, dst_hbm, out_hbm, t_v, acc_v,
             (src_v0, src_v1), (dst_v0, dst_v1), (sem0, sem1),
             wid, base, NCH_E)


def _sc_ac(srcp, dstp, tatc):
    fn = pl.kernel(
        _ac_body,
        out_type=jax.ShapeDtypeStruct((NW * NPAD,), _f32),
        mesh=_mesh,
        compiler_params=_sc_params,
        scratch_types=[
            pltpu.VMEM((NPAD,), _f32),
            pltpu.VMEM((NPAD,), _f32),
            pltpu.VMEM((CH,), jnp.int32),
            pltpu.VMEM((CH,), jnp.int32),
            pltpu.VMEM((CH,), jnp.int32),
            pltpu.VMEM((CH,), jnp.int32),
            pltpu.SemaphoreType.DMA,
            pltpu.SemaphoreType.DMA,
            pltpu.SemaphoreType.DMA,
        ],
    )
    return fn(srcp, dstp, tatc)


def _pool_body(part_hbm, tatc_hbm, dinv_hbm, batch_hbm, out_hbm,
               p_v, ta_v, tc_v, dinv_v, b_v, accA, accC, accN,
               semp, semm):
    wid = _wid()
    lo = wid * SL
    zs = jnp.zeros((16,), dtype=_f32)

    @pl.loop(0, NW)
    def _ld(r):
        pltpu.async_copy(part_hbm.at[pl.ds(r * NPAD + lo, SL)],
                         p_v.at[pl.ds(r * SL, SL)], semp)

    pltpu.async_copy(tatc_hbm.at[pl.ds(lo, SL)], ta_v, semm)
    pltpu.async_copy(tatc_hbm.at[pl.ds(NPAD + lo, SL)], tc_v, semm)
    pltpu.async_copy(dinv_hbm.at[pl.ds(lo, SL)], dinv_v, semm)
    pltpu.async_copy(batch_hbm.at[pl.ds(lo, SL)], b_v, semm)

    @pl.loop(0, GP, step=16)
    def _z(i):
        accA[pl.ds(i, 16)] = zs
        accC[pl.ds(i, 16)] = zs
        accN[pl.ds(i, 16)] = zs

    @pl.loop(0, NW)
    def _wt(r):
        pltpu.make_async_copy(part_hbm.at[pl.ds(lo, SL)],
                              p_v.at[pl.ds(0, SL)], semp).wait()

    pltpu.make_async_copy(dinv_hbm.at[pl.ds(lo, SL)], ta_v, semm).wait()
    pltpu.make_async_copy(dinv_hbm.at[pl.ds(lo, SL)], tc_v, semm).wait()
    pltpu.make_async_copy(dinv_hbm.at[pl.ds(lo, SL)], dinv_v, semm).wait()
    pltpu.make_async_copy(batch_hbm.at[pl.ds(lo, SL)], b_v, semm).wait()
    ones = jnp.full((16,), 1.0, dtype=_f32)

    @pl.loop(0, SL, step=16)
    def _inner(i):
        sl = pl.ds(i, 16)
        Ap = p_v[pl.ds(i, 16)]
        Cp = p_v[pl.ds(16 * SL + i, 16)]
        for r in range(1, 16):
            Ap = Ap + p_v[pl.ds(r * SL + i, 16)]
            Cp = Cp + p_v[pl.ds((16 + r) * SL + i, 16)]
        dv = dinv_v[sl]
        Av = dv * (Ap + ta_v[sl])
        Cv = dv * (Cp + tc_v[sl])
        bi = b_v[sl]
        plsc.addupdate_scatter(accA, [bi], Av)
        plsc.addupdate_scatter(accC, [bi], Cv)
        plsc.addupdate_scatter(accN, [bi], ones)

    base3 = wid * 3 * GP
    pltpu.sync_copy(accA, out_hbm.at[pl.ds(base3, GP)])
    pltpu.sync_copy(accC, out_hbm.at[pl.ds(base3 + GP, GP)])
    pltpu.sync_copy(accN, out_hbm.at[pl.ds(base3 + 2 * GP, GP)])


def _sc_pool(part, tatc, dinv1d, batchp):
    fn = pl.kernel(
        _pool_body,
        out_type=jax.ShapeDtypeStruct((NW * 3 * GP,), _f32),
        mesh=_mesh,
        compiler_params=_sc_params,
        scratch_types=[
            pltpu.VMEM((NW * SL,), _f32),
            pltpu.VMEM((SL,), _f32),
            pltpu.VMEM((SL,), _f32),
            pltpu.VMEM((SL,), _f32),
            pltpu.VMEM((SL,), jnp.int32),
            pltpu.VMEM((GP,), _f32),
            pltpu.VMEM((GP,), _f32),
            pltpu.VMEM((GP,), _f32),
            pltpu.SemaphoreType.DMA,
            pltpu.SemaphoreType.DMA,
        ],
    )
    return fn(part, tatc, dinv1d, batchp)


def _tcB_body(part_ref, x_ref, dinv_ref, t_ref):
    deg = jnp.sum(part_ref[...], axis=0) + 1.0
    dinv = lax.rsqrt(deg)
    dinv_ref[...] = dinv
    t_ref[...] = x_ref[...] * dinv


def _tc_b(part3, x2):
    return pl.pallas_call(
        _tcB_body,
        out_shape=[
            jax.ShapeDtypeStruct((NR, 128), _f32),
            jax.ShapeDtypeStruct((NR, 128), _f32),
        ],
    )(part3, x2)


def _tcD_body(part_ref, t_ref, dinv_ref, tatc_ref):
    sp = jnp.sum(part_ref[...], axis=0)
    dinv = dinv_ref[...]
    s = dinv * (sp + t_ref[...])
    tatc_ref[0] = dinv * jnp.maximum(s, 0.0)
    tatc_ref[1] = dinv * jnp.maximum(-s, 0.0)


def _tc_d(part3, t2, dinv2):
    return pl.pallas_call(
        _tcD_body,
        out_shape=jax.ShapeDtypeStruct((2, NR, 128), _f32),
    )(part3, t2, dinv2)


def _dot(a, b, dims):
    return lax.dot_general(a, b, (dims, ((), ())),
                           precision=lax.Precision.HIGHEST,
                           preferred_element_type=_f32)


def _tcF_body(pp_ref, W1_ref, W2_ref, Wc1_ref, Wc2_ref, b2_ref, bc1_ref,
              bc2_ref, out_ref):
    p3 = jnp.sum(pp_ref[...], axis=0)
    sumA = p3[0:1, 0:G]
    sumC = p3[1:2, 0:G]
    cnt = p3[2:3, 0:G]
    invc = 1.0 / jnp.maximum(cnt, 1.0)
    Abar = sumA * invc
    Cbar = sumC * invc
    m = jnp.minimum(cnt, 1.0)

    u = jnp.maximum(W1_ref[...], 0.0)
    v = jnp.maximum(-W1_ref[...], 0.0)
    p = _dot(u, W2_ref[...], ((1,), (0,)))
    q = _dot(v, W2_ref[...], ((1,), (0,)))
    Pc = _dot(Wc1_ref[...], p, ((0,), (1,)))
    Qc = _dot(Wc1_ref[...], q, ((0,), (1,)))
    Rc = _dot(Wc1_ref[...], b2_ref[...], ((0,), (1,)))
    zT = Pc * Abar + Qc * Cbar + Rc * m + bc1_ref[...]
    zT = jnp.maximum(zT, 0.0)
    o = _dot(Wc2_ref[...], zT, ((0,), (0,)))
    o = o + bc2_ref[...]
    out_ref[...] = 1.0 / (1.0 + jnp.exp(-o))


def _tc_f(pp, W1, W2, Wc1, Wc2, b2, bc1, bc2):
    return pl.pallas_call(
        _tcF_body,
        out_shape=jax.ShapeDtypeStruct((1, G), _f32),
    )(pp, W1, W2, Wc1, Wc2, b2.reshape(1, H), bc1.reshape(32, 1),
      bc2.reshape(1, 1))


def kernel(x, edge_index, batch, W1, b1, W2, b2, Wc1, bc1, Wc2, bc2):
    i32 = jnp.int32
    pad = jnp.full((E_PAD - E,), N, i32)
    srcp = jnp.concatenate([edge_index[0].astype(i32), pad])
    dstp = jnp.concatenate([edge_index[1].astype(i32), pad])
    xpad = jnp.pad(x[:, 0], (0, NPAD - N))
    batchp = jnp.pad(batch.astype(i32), (0, NPAD - N), constant_values=G)
    degp = _sc_deg(dstp)
    dinv2, t2 = _tc_b(degp.reshape(NW, NR, 128), xpad.reshape(NR, 128))
    t1d = t2.reshape(NPAD)
    dinv1d = dinv2.reshape(NPAD)

    spp = _sc_gs(srcp, dstp, t1d)
    tatc3 = _tc_d(spp.reshape(NW, NR, 128), t2, dinv2)
    tatc = tatc3.reshape(2 * NPAD)

    acp = _sc_ac(srcp, dstp, tatc)
    pooled = _sc_pool(acp, tatc, dinv1d, batchp)

    out = _tc_f(pooled.reshape(NW, 3, GP), W1, W2, Wc1, Wc2, b2, bc1, bc2)
    return out.reshape(G, 1)


# speedup vs baseline: 1.0755x; 1.0555x over previous
import dataclasses
import functools

import jax
import jax.numpy as jnp
from jax import lax
from jax.experimental import pallas as pl
from jax.experimental.pallas import tpu as pltpu
from jax.experimental.pallas import tpu_sc as plsc

N = 50000
E = 800000
H = 64
G = 512

NW = 32
NPAD = 50176
NR = 392
SL = NPAD // NW
E_PAD = 802816
CH = 6272
W_AC = E_PAD // NW
NCH_AC = W_AC // CH
W_E = E_PAD // 16
NCH_E = W_E // CH
GP = 640

_mesh = plsc.VectorSubcoreMesh(core_axis_name="c", subcore_axis_name="s")
_f32 = jnp.float32

_sc_params = pltpu.CompilerParams()
if "needs_layout_passes" in pltpu.CompilerParams.__dataclass_fields__:
    _sc_params = dataclasses.replace(_sc_params, needs_layout_passes=False)


def _wid():
    return lax.axis_index("c") * 16 + lax.axis_index("s")


def _deg_body(dst_hbm, zeros_hbm, out_hbm, acc_v, dst_v0, dst_v1,
              semz, sem0, sem1):
    wid = _wid()
    base = wid * W_AC
    ones = jnp.full((16,), 1.0, dtype=_f32)
    bufs = (dst_v0, dst_v1)
    sems = (sem0, sem1)

    def start(b, off):
        pltpu.async_copy(dst_hbm.at[pl.ds(off, CH)], bufs[b], sems[b])

    def wait(b):
        pltpu.make_async_copy(dst_hbm.at[pl.ds(0, CH)], bufs[b],
                              sems[b]).wait()

    def scat(b):
        @pl.loop(0, CH, step=128)
        def _inner(i):
            for u in range(8):
                di = bufs[b][pl.ds(i + u * 16, 16)]
                plsc.addupdate_scatter(acc_v, [di], ones)

    pltpu.async_copy(zeros_hbm, acc_v, semz)
    start(0, base)
    pltpu.make_async_copy(zeros_hbm, acc_v, semz).wait()

    @pl.loop(0, NCH_AC, step=2)
    def _chunks(j):
        start(1, base + (j + 1) * CH)
        wait(0)
        scat(0)

        @pl.when(j + 2 < NCH_AC)
        def _():
            start(0, base + (j + 2) * CH)

        wait(1)
        scat(1)

    pltpu.sync_copy(acc_v, out_hbm.at[pl.ds(wid * NPAD, NPAD)])


def _sc_deg(dstp, zeros):
    fn = pl.kernel(
        _deg_body,
        out_type=jax.ShapeDtypeStruct((NW * NPAD,), _f32),
        mesh=_mesh,
        compiler_params=_sc_params,
        scratch_types=[
            pltpu.VMEM((NPAD,), _f32),
            pltpu.VMEM((CH,), jnp.int32),
            pltpu.VMEM((CH,), jnp.int32),
            pltpu.SemaphoreType.DMA,
            pltpu.SemaphoreType.DMA,
            pltpu.SemaphoreType.DMA,
        ],
    )
    return fn(dstp, zeros)


def _gs_loop(src_hbm, dst_hbm, out_hbm, t_v, acc_v, sbufs, dbufs, sems,
             wid, base, nch):
    def start(b, off):
        pltpu.async_copy(src_hbm.at[pl.ds(off, CH)], sbufs[b], sems[b])
        pltpu.async_copy(dst_hbm.at[pl.ds(off, CH)], dbufs[b], sems[b])

    def wait(b):
        pltpu.make_async_copy(src_hbm.at[pl.ds(0, CH)], sbufs[b],
                              sems[b]).wait()
        pltpu.make_async_copy(dst_hbm.at[pl.ds(0, CH)], dbufs[b],
                              sems[b]).wait()

    def scat(b):
        @pl.loop(0, CH, step=128)
        def _inner(i):
            for u in range(8):
                si = sbufs[b][pl.ds(i + u * 16, 16)]
                di = dbufs[b][pl.ds(i + u * 16, 16)]
                vals = plsc.load_gather(t_v, [si])
                plsc.addupdate_scatter(acc_v, [di], vals)

    @pl.loop(0, nch, step=2)
    def _chunks(j):
        start(1, base + (j + 1) * CH)
        wait(0)
        scat(0)

        @pl.when(j + 2 < nch)
        def _():
            start(0, base + (j + 2) * CH)

        wait(1)
        scat(1)

    pltpu.sync_copy(acc_v, out_hbm.at[pl.ds(wid * NPAD, NPAD)])


def _gs_body(src_hbm, dst_hbm, t_hbm, zeros_hbm, out_hbm,
             t_v, acc_v, src_v0, dst_v0, src_v1, dst_v1, semt, sem0, sem1):
    wid = _wid()
    base = wid * W_AC
    pltpu.async_copy(t_hbm, t_v, semt)
    pltpu.async_copy(zeros_hbm, acc_v, semt)
    pltpu.async_copy(src_hbm.at[pl.ds(base, CH)], src_v0, sem0)
    pltpu.async_copy(dst_hbm.at[pl.ds(base, CH)], dst_v0, sem0)
    pltpu.make_async_copy(t_hbm, t_v, semt).wait()
    pltpu.make_async_copy(zeros_hbm, acc_v, semt).wait()
    _gs_loop(src_hbm, dst_hbm, out_hbm, t_v, acc_v,
             (src_v0, src_v1), (dst_v0, dst_v1), (sem0, sem1),
             wid, base, NCH_AC)


def _sc_gs(srcp, dstp, t1d, zeros):
    fn = pl.kernel(
        _gs_body,
        out_type=jax.ShapeDtypeStruct((NW * NPAD,), _f32),
        mesh=_mesh,
        compiler_params=_sc_params,
        scratch_types=[
            pltpu.VMEM((NPAD,), _f32),
            pltpu.VMEM((NPAD,), _f32),
            pltpu.VMEM((CH,), jnp.int32),
            pltpu.VMEM((CH,), jnp.int32),
            pltpu.VMEM((CH,), jnp.int32),
            pltpu.VMEM((CH,), jnp.int32),
            pltpu.SemaphoreType.DMA,
            pltpu.SemaphoreType.DMA,
            pltpu.SemaphoreType.DMA,
        ],
    )
    return fn(srcp, dstp, t1d, zeros)


def _ac_body(src_hbm, dst_hbm, tatc_hbm, zeros_hbm, out_hbm,
             t_v, acc_v, src_v0, dst_v0, src_v1, dst_v1, semt, sem0, sem1):
    c = lax.axis_index("c")
    s = lax.axis_index("s")
    wid = c * 16 + s
    base = s * W_E
    pltpu.async_copy(tatc_hbm.at[pl.ds(c * NPAD, NPAD)], t_v, semt)
    pltpu.async_copy(zeros_hbm, acc_v, semt)
    pltpu.async_copy(src_hbm.at[pl.ds(base, CH)], src_v0, sem0)
    pltpu.async_copy(dst_hbm.at[pl.ds(base, CH)], dst_v0, sem0)
    pltpu.make_async_copy(tatc_hbm.at[pl.ds(0, NPAD)], t_v, semt).wait()
    pltpu.make_async_copy(zeros_hbm, acc_v, semt).wait()
    _gs_loop(src_hbm, dst_hbm, out_hbm, t_v, acc_v,
             (src_v0, src_v1), (dst_v0, dst_v1), (sem0, sem1),
             wid, base, NCH_E)


def _sc_ac(srcp, dstp, tatc, zeros):
    fn = pl.kernel(
        _ac_body,
        out_type=jax.ShapeDtypeStruct((NW * NPAD,), _f32),
        mesh=_mesh,
        compiler_params=_sc_params,
        scratch_types=[
            pltpu.VMEM((NPAD,), _f32),
            pltpu.VMEM((NPAD,), _f32),
            pltpu.VMEM((CH,), jnp.int32),
            pltpu.VMEM((CH,), jnp.int32),
            pltpu.VMEM((CH,), jnp.int32),
            pltpu.VMEM((CH,), jnp.int32),
            pltpu.SemaphoreType.DMA,
            pltpu.SemaphoreType.DMA,
            pltpu.SemaphoreType.DMA,
        ],
    )
    return fn(srcp, dstp, tatc, zeros)


def _pool_body(part_hbm, tatc_hbm, dinv_hbm, batch_hbm, out_hbm,
               p_v, ta_v, tc_v, dinv_v, b_v, accA, accC, accN,
               semp, semm):
    wid = _wid()
    lo = wid * SL
    zs = jnp.zeros((16,), dtype=_f32)

    @pl.loop(0, NW)
    def _ld(r):
        pltpu.async_copy(part_hbm.at[pl.ds(r * NPAD + lo, SL)],
                         p_v.at[pl.ds(r * SL, SL)], semp)

    pltpu.async_copy(tatc_hbm.at[pl.ds(lo, SL)], ta_v, semm)
    pltpu.async_copy(tatc_hbm.at[pl.ds(NPAD + lo, SL)], tc_v, semm)
    pltpu.async_copy(dinv_hbm.at[pl.ds(lo, SL)], dinv_v, semm)
    pltpu.async_copy(batch_hbm.at[pl.ds(lo, SL)], b_v, semm)

    @pl.loop(0, GP, step=16)
    def _z(i):
        accA[pl.ds(i, 16)] = zs
        accC[pl.ds(i, 16)] = zs
        accN[pl.ds(i, 16)] = zs

    @pl.loop(0, NW)
    def _wt(r):
        pltpu.make_async_copy(part_hbm.at[pl.ds(lo, SL)],
                              p_v.at[pl.ds(0, SL)], semp).wait()

    pltpu.make_async_copy(dinv_hbm.at[pl.ds(lo, SL)], ta_v, semm).wait()
    pltpu.make_async_copy(dinv_hbm.at[pl.ds(lo, SL)], tc_v, semm).wait()
    pltpu.make_async_copy(dinv_hbm.at[pl.ds(lo, SL)], dinv_v, semm).wait()
    pltpu.make_async_copy(batch_hbm.at[pl.ds(lo, SL)], b_v, semm).wait()
    ones = jnp.full((16,), 1.0, dtype=_f32)

    @pl.loop(0, SL, step=16)
    def _inner(i):
        sl = pl.ds(i, 16)
        Ap = p_v[pl.ds(i, 16)]
        Cp = p_v[pl.ds(16 * SL + i, 16)]
        for r in range(1, 16):
            Ap = Ap + p_v[pl.ds(r * SL + i, 16)]
            Cp = Cp + p_v[pl.ds((16 + r) * SL + i, 16)]
        dv = dinv_v[sl]
        Av = dv * (Ap + ta_v[sl])
        Cv = dv * (Cp + tc_v[sl])
        bi = b_v[sl]
        plsc.addupdate_scatter(accA, [bi], Av)
        plsc.addupdate_scatter(accC, [bi], Cv)
        plsc.addupdate_scatter(accN, [bi], ones)

    base3 = wid * 3 * GP
    pltpu.sync_copy(accA, out_hbm.at[pl.ds(base3, GP)])
    pltpu.sync_copy(accC, out_hbm.at[pl.ds(base3 + GP, GP)])
    pltpu.sync_copy(accN, out_hbm.at[pl.ds(base3 + 2 * GP, GP)])


def _sc_pool(part, tatc, dinv1d, batchp):
    fn = pl.kernel(
        _pool_body,
        out_type=jax.ShapeDtypeStruct((NW * 3 * GP,), _f32),
        mesh=_mesh,
        compiler_params=_sc_params,
        scratch_types=[
            pltpu.VMEM((NW * SL,), _f32),
            pltpu.VMEM((SL,), _f32),
            pltpu.VMEM((SL,), _f32),
            pltpu.VMEM((SL,), _f32),
            pltpu.VMEM((SL,), jnp.int32),
            pltpu.VMEM((GP,), _f32),
            pltpu.VMEM((GP,), _f32),
            pltpu.VMEM((GP,), _f32),
            pltpu.SemaphoreType.DMA,
            pltpu.SemaphoreType.DMA,
        ],
    )
    return fn(part, tatc, dinv1d, batchp)


def _tcB_body(part_ref, x_ref, dinv_ref, t_ref):
    deg = jnp.sum(part_ref[...], axis=0) + 1.0
    dinv = lax.rsqrt(deg)
    dinv_ref[...] = dinv
    t_ref[...] = x_ref[...] * dinv


def _tc_b(part3, x2):
    return pl.pallas_call(
        _tcB_body,
        out_shape=[
            jax.ShapeDtypeStruct((NR, 128), _f32),
            jax.ShapeDtypeStruct((NR, 128), _f32),
        ],
    )(part3, x2)


def _tcD_body(part_ref, t_ref, dinv_ref, tatc_ref):
    sp = jnp.sum(part_ref[...], axis=0)
    dinv = dinv_ref[...]
    s = dinv * (sp + t_ref[...])
    tatc_ref[0] = dinv * jnp.maximum(s, 0.0)
    tatc_ref[1] = dinv * jnp.maximum(-s, 0.0)


def _tc_d(part3, t2, dinv2):
    return pl.pallas_call(
        _tcD_body,
        out_shape=jax.ShapeDtypeStruct((2, NR, 128), _f32),
    )(part3, t2, dinv2)


def _dot(a, b, dims):
    return lax.dot_general(a, b, (dims, ((), ())),
                           precision=lax.Precision.HIGHEST,
                           preferred_element_type=_f32)


def _tcF_body(pp_ref, W1_ref, W2_ref, Wc1_ref, Wc2_ref, b2_ref, bc1_ref,
              bc2_ref, out_ref):
    p3 = jnp.sum(pp_ref[...], axis=0)
    sumA = p3[0:1, 0:G]
    sumC = p3[1:2, 0:G]
    cnt = p3[2:3, 0:G]
    invc = 1.0 / jnp.maximum(cnt, 1.0)
    Abar = sumA * invc
    Cbar = sumC * invc
    m = jnp.minimum(cnt, 1.0)

    u = jnp.maximum(W1_ref[...], 0.0)
    v = jnp.maximum(-W1_ref[...], 0.0)
    p = _dot(u, W2_ref[...], ((1,), (0,)))
    q = _dot(v, W2_ref[...], ((1,), (0,)))
    Pc = _dot(Wc1_ref[...], p, ((0,), (1,)))
    Qc = _dot(Wc1_ref[...], q, ((0,), (1,)))
    Rc = _dot(Wc1_ref[...], b2_ref[...], ((0,), (1,)))
    zT = Pc * Abar + Qc * Cbar + Rc * m + bc1_ref[...]
    zT = jnp.maximum(zT, 0.0)
    o = _dot(Wc2_ref[...], zT, ((0,), (0,)))
    o = o + bc2_ref[...]
    out_ref[...] = 1.0 / (1.0 + jnp.exp(-o))


def _tc_f(pp, W1, W2, Wc1, Wc2, b2, bc1, bc2):
    return pl.pallas_call(
        _tcF_body,
        out_shape=jax.ShapeDtypeStruct((1, G), _f32),
    )(pp, W1, W2, Wc1, Wc2, b2.reshape(1, H), bc1.reshape(32, 1),
      bc2.reshape(1, 1))


def kernel(x, edge_index, batch, W1, b1, W2, b2, Wc1, bc1, Wc2, bc2):
    i32 = jnp.int32
    pad = jnp.full((E_PAD - E,), N, i32)
    srcp = jnp.concatenate([edge_index[0].astype(i32), pad])
    dstp = jnp.concatenate([edge_index[1].astype(i32), pad])
    xpad = jnp.pad(x[:, 0], (0, NPAD - N))
    batchp = jnp.pad(batch.astype(i32), (0, NPAD - N), constant_values=G)
    zeros = jnp.zeros((NPAD,), _f32)

    degp = _sc_deg(dstp, zeros)
    dinv2, t2 = _tc_b(degp.reshape(NW, NR, 128), xpad.reshape(NR, 128))
    t1d = t2.reshape(NPAD)
    dinv1d = dinv2.reshape(NPAD)

    spp = _sc_gs(srcp, dstp, t1d, zeros)
    tatc3 = _tc_d(spp.reshape(NW, NR, 128), t2, dinv2)
    tatc = tatc3.reshape(2 * NPAD)

    acp = _sc_ac(srcp, dstp, tatc, zeros)
    pooled = _sc_pool(acp, tatc, dinv1d, batchp)

    out = _tc_f(pooled.reshape(NW, 3, GP), W1, W2, Wc1, Wc2, b2, bc1, bc2)
    return out.reshape(G, 1)
